# Initial kernel scaffold; baseline (speedup 1.0000x reference)
import functools

import jax
import jax.numpy as jnp
from jax import lax
from jax.experimental import pallas as pl
from jax.experimental.pallas import tpu as pltpu
from jax.experimental.pallas import tpu_sc as plsc

N = 10000
D = 128
E = 320000
NC = 2
NS = 16
NW = NC * NS
CHUNK = 128
E_TOT = E + N
CPT = -(-E_TOT // (NW * CHUNK))
E_PAD = NW * CPT * CHUNK
DUMP = N
N_PAD = 10368
RPT = N_PAD // NS

_MESH = plsc.VectorSubcoreMesh(core_axis_name="c", subcore_axis_name="s")


def _hist_body(col_hbm, z16_hbm, ones_hbm, outa, outb, dacc, idx_c, ones_v, sem):
    c = lax.axis_index("c")
    s = lax.axis_index("s")
    w = c * NS + s
    pltpu.sync_copy(col_hbm.at[w], idx_c)
    pltpu.sync_copy(ones_hbm, ones_v)
    r0 = s * RPT
    pltpu.sync_copy(z16_hbm.at[pl.ds(r0, RPT)], dacc.at[pl.ds(r0, RPT)])
    plsc.subcore_barrier()

    def body(j, carry):
        pltpu.sync_copy(ones_v, dacc.at[idx_c.at[j]], add=True)
        return carry

    lax.fori_loop(0, CPT, body, 0)
    plsc.subcore_barrier()

    @pl.when(c == 0)
    def _():
        pltpu.sync_copy(dacc.at[pl.ds(r0, RPT)], outa.at[pl.ds(r0, RPT)])

    @pl.when(c == 1)
    def _():
        pltpu.sync_copy(dacc.at[pl.ds(r0, RPT)], outb.at[pl.ds(r0, RPT)])


_hist = pl.kernel(
    _hist_body,
    out_type=(
        jax.ShapeDtypeStruct((N_PAD, 16), jnp.float32),
        jax.ShapeDtypeStruct((N_PAD, 16), jnp.float32),
    ),
    mesh=_MESH,
    scratch_types=[
        pltpu.VMEM_SHARED((N_PAD, 16), jnp.float32),
        pltpu.VMEM((CPT, CHUNK), jnp.int32),
        pltpu.VMEM((CHUNK, 16), jnp.float32),
        pltpu.SemaphoreType.DMA,
    ],
)


def _hop_body(g_hbm, row_hbm, col_hbm, z_hbm, outa, outb, acc, idx_r, idx_c,
              rows_v, sem):
    c = lax.axis_index("c")
    s = lax.axis_index("s")
    w = c * NS + s
    pltpu.sync_copy(row_hbm.at[w], idx_r)
    pltpu.sync_copy(col_hbm.at[w], idx_c)
    r0 = s * RPT
    pltpu.sync_copy(z_hbm.at[pl.ds(r0, RPT)], acc.at[pl.ds(r0, RPT)])
    plsc.subcore_barrier()

    def body(j, carry):
        pltpu.async_copy(g_hbm.at[idx_r.at[j]], rows_v, sem).wait()
        pltpu.sync_copy(rows_v, acc.at[idx_c.at[j]], add=True)
        return carry

    lax.fori_loop(0, CPT, body, 0)
    plsc.subcore_barrier()

    @pl.when(c == 0)
    def _():
        pltpu.sync_copy(acc.at[pl.ds(r0, RPT)], outa.at[pl.ds(r0, RPT)])

    @pl.when(c == 1)
    def _():
        pltpu.sync_copy(acc.at[pl.ds(r0, RPT)], outb.at[pl.ds(r0, RPT)])


_hop = pl.kernel(
    _hop_body,
    out_type=(
        jax.ShapeDtypeStruct((N_PAD, D), jnp.float32),
        jax.ShapeDtypeStruct((N_PAD, D), jnp.float32),
    ),
    mesh=_MESH,
    scratch_types=[
        pltpu.VMEM_SHARED((N_PAD, D), jnp.float32),
        pltpu.VMEM((CPT, CHUNK), jnp.int32),
        pltpu.VMEM((CPT, CHUNK), jnp.int32),
        pltpu.VMEM((CHUNK, D), jnp.float32),
        pltpu.SemaphoreType.DMA,
    ],
)


def _dinv_block(dga_ref, dgb_ref):
    deg = dga_ref[:, 0] + dgb_ref[:, 0]
    return jnp.where(deg > 0, 1.0 / jnp.sqrt(deg), 0.0)


def _scale0_body(dga_ref, dgb_ref, x_ref, o_ref):
    dinv = _dinv_block(dga_ref, dgb_ref)
    o_ref[...] = x_ref[...] * dinv[:, None]


def _mid_body(dga_ref, dgb_ref, pa_ref, pb_ref, o_ref):
    dinv = _dinv_block(dga_ref, dgb_ref)
    o_ref[...] = (pa_ref[...] + pb_ref[...]) * (dinv * dinv)[:, None]


def _final_body(dga_ref, dgb_ref, pa_ref, pb_ref, w_ref, b_ref, o_ref):
    dinv = _dinv_block(dga_ref, dgb_ref)
    h = (pa_ref[...] + pb_ref[...]) * dinv[:, None]
    z = jnp.dot(h, w_ref[...], preferred_element_type=jnp.float32) + b_ref[...]
    m = jnp.max(z, axis=1, keepdims=True)
    zz = z - m
    lse = jnp.log(jnp.sum(jnp.exp(zz), axis=1, keepdims=True))
    o_ref[...] = zz - lse


_BR = 576
_deg_spec = pl.BlockSpec((_BR, 16), lambda i: (i, 0))
_row_spec = pl.BlockSpec((_BR, D), lambda i: (i, 0))

_scale0 = pl.pallas_call(
    _scale0_body,
    grid=(N_PAD // _BR,),
    in_specs=[_deg_spec, _deg_spec, _row_spec],
    out_specs=_row_spec,
    out_shape=jax.ShapeDtypeStruct((N_PAD, D), jnp.float32),
)

_mid = pl.pallas_call(
    _mid_body,
    grid=(N_PAD // _BR,),
    in_specs=[_deg_spec, _deg_spec, _row_spec, _row_spec],
    out_specs=_row_spec,
    out_shape=jax.ShapeDtypeStruct((N_PAD, D), jnp.float32),
)

_BF = 400
_final = pl.pallas_call(
    _final_body,
    grid=(N // _BF,),
    in_specs=[
        pl.BlockSpec((_BF, 16), lambda i: (i, 0)),
        pl.BlockSpec((_BF, 16), lambda i: (i, 0)),
        pl.BlockSpec((_BF, D), lambda i: (i, 0)),
        pl.BlockSpec((_BF, D), lambda i: (i, 0)),
        pl.BlockSpec((D, D), lambda i: (0, 0)),
        pl.BlockSpec((1, D), lambda i: (0, 0)),
    ],
    out_specs=pl.BlockSpec((_BF, D), lambda i: (i, 0)),
    out_shape=jax.ShapeDtypeStruct((N, D), jnp.float32),
)


def kernel(x, edge_index, W, b):
    loops = jnp.arange(N, dtype=jnp.int32)
    pad = jnp.full((E_PAD - E_TOT,), DUMP, dtype=jnp.int32)
    row = jnp.concatenate([edge_index[0], loops, pad]).reshape(NW, CPT, CHUNK)
    col = jnp.concatenate([edge_index[1], loops, pad]).reshape(NW, CPT, CHUNK)
    x_pad = jnp.pad(x, ((0, N_PAD - N), (0, 0)))
    z128 = jnp.zeros((N_PAD, D), jnp.float32)
    z16 = jnp.zeros((N_PAD, 16), jnp.float32)
    ones16 = jnp.ones((CHUNK, 16), jnp.float32)

    dga, dgb = _hist(col, z16, ones16)
    g0 = _scale0(dga, dgb, x_pad)
    p1a, p1b = _hop(g0, row, col, z128)
    g1 = _mid(dga, dgb, p1a, p1b)
    p2a, p2b = _hop(g1, row, col, z128)
    return _final(dga, dgb, p2a, p2b, W, b.reshape(1, D))


# speedup vs baseline: 14.7436x; 14.7436x over previous
import functools

import jax
import jax.numpy as jnp
from jax import lax
from jax.experimental import pallas as pl
from jax.experimental.pallas import tpu as pltpu
from jax.experimental.pallas import tpu_sc as plsc

N = 10000
D = 128
E = 320000
NC = 2
NS = 16
NW = NC * NS
CHUNK = 128
E_TOT = E + N
CPT = -(-E_TOT // (NW * CHUNK))
E_PAD = NW * CPT * CHUNK
DUMP = N
N_PAD = 10368
RPT = N_PAD // NS

_MESH = plsc.VectorSubcoreMesh(core_axis_name="c", subcore_axis_name="s")


def _hist_body(col_hbm, z_hbm, ones_hbm, out, dacc, idx_c, ones_v, sem):
    c = lax.axis_index("c")
    s = lax.axis_index("s")
    w = c * NS + s
    pltpu.sync_copy(col_hbm.at[w], idx_c)
    pltpu.sync_copy(ones_hbm, ones_v)
    r0 = s * RPT
    pltpu.sync_copy(z_hbm.at[pl.ds(r0, RPT)], dacc.at[pl.ds(r0, RPT)])
    plsc.subcore_barrier()

    def body(j, carry):
        pltpu.sync_copy(ones_v, dacc.at[idx_c.at[j]], add=True)
        return carry

    lax.fori_loop(0, CPT, body, 0)
    plsc.subcore_barrier()
    pltpu.sync_copy(dacc.at[pl.ds(r0, RPT)], out.at[c, pl.ds(r0, RPT)])


_hist = pl.kernel(
    _hist_body,
    out_type=jax.ShapeDtypeStruct((NC, N_PAD, D), jnp.float32),
    mesh=_MESH,
    scratch_types=[
        pltpu.VMEM_SHARED((N_PAD, D), jnp.float32),
        pltpu.VMEM((CPT, CHUNK), jnp.int32),
        pltpu.VMEM((CHUNK, D), jnp.float32),
        pltpu.SemaphoreType.DMA,
    ],
)


def _hop_body(g_hbm, row_hbm, col_hbm, z_hbm, out, acc, idx_r, idx_c,
              rows_v, sem):
    c = lax.axis_index("c")
    s = lax.axis_index("s")
    w = c * NS + s
    pltpu.sync_copy(row_hbm.at[w], idx_r)
    pltpu.sync_copy(col_hbm.at[w], idx_c)
    r0 = s * RPT
    pltpu.sync_copy(z_hbm.at[pl.ds(r0, RPT)], acc.at[pl.ds(r0, RPT)])
    plsc.subcore_barrier()

    def body(j, carry):
        pltpu.async_copy(g_hbm.at[idx_r.at[j]], rows_v, sem).wait()
        pltpu.sync_copy(rows_v, acc.at[idx_c.at[j]], add=True)
        return carry

    lax.fori_loop(0, CPT, body, 0)
    plsc.subcore_barrier()
    pltpu.sync_copy(acc.at[pl.ds(r0, RPT)], out.at[c, pl.ds(r0, RPT)])


_hop = pl.kernel(
    _hop_body,
    out_type=jax.ShapeDtypeStruct((NC, N_PAD, D), jnp.float32),
    mesh=_MESH,
    scratch_types=[
        pltpu.VMEM_SHARED((N_PAD, D), jnp.float32),
        pltpu.VMEM((CPT, CHUNK), jnp.int32),
        pltpu.VMEM((CPT, CHUNK), jnp.int32),
        pltpu.VMEM((CHUNK, D), jnp.float32),
        pltpu.SemaphoreType.DMA,
    ],
)


def _dinv_block(dg_ref):
    deg = dg_ref[0, :, 0] + dg_ref[1, :, 0]
    return jnp.where(deg > 0, 1.0 / jnp.sqrt(deg), 0.0)


def _scale0_body(dg_ref, x_ref, o_ref):
    dinv = _dinv_block(dg_ref)
    o_ref[...] = x_ref[...] * dinv[:, None]


def _mid_body(dg_ref, p_ref, o_ref):
    dinv = _dinv_block(dg_ref)
    o_ref[...] = (p_ref[0] + p_ref[1]) * (dinv * dinv)[:, None]


def _final_body(dg_ref, p_ref, w_ref, b_ref, o_ref):
    dinv = _dinv_block(dg_ref)
    h = (p_ref[0] + p_ref[1]) * dinv[:, None]
    z = jnp.dot(h, w_ref[...], preferred_element_type=jnp.float32) + b_ref[...]
    m = jnp.max(z, axis=1, keepdims=True)
    zz = z - m
    lse = jnp.log(jnp.sum(jnp.exp(zz), axis=1, keepdims=True))
    o_ref[...] = zz - lse


_BR = 576
_deg_spec = pl.BlockSpec((NC, _BR, D), lambda i: (0, i, 0))
_row_spec = pl.BlockSpec((_BR, D), lambda i: (i, 0))
_p_spec = pl.BlockSpec((NC, _BR, D), lambda i: (0, i, 0))

_scale0 = pl.pallas_call(
    _scale0_body,
    grid=(N_PAD // _BR,),
    in_specs=[_deg_spec, _row_spec],
    out_specs=_row_spec,
    out_shape=jax.ShapeDtypeStruct((N_PAD, D), jnp.float32),
)

_mid = pl.pallas_call(
    _mid_body,
    grid=(N_PAD // _BR,),
    in_specs=[_deg_spec, _p_spec],
    out_specs=_row_spec,
    out_shape=jax.ShapeDtypeStruct((N_PAD, D), jnp.float32),
)

_BF = 400
_final = pl.pallas_call(
    _final_body,
    grid=(N // _BF,),
    in_specs=[
        pl.BlockSpec((NC, _BF, D), lambda i: (0, i, 0)),
        pl.BlockSpec((NC, _BF, D), lambda i: (0, i, 0)),
        pl.BlockSpec((D, D), lambda i: (0, 0)),
        pl.BlockSpec((1, D), lambda i: (0, 0)),
    ],
    out_specs=pl.BlockSpec((_BF, D), lambda i: (i, 0)),
    out_shape=jax.ShapeDtypeStruct((N, D), jnp.float32),
)


def kernel(x, edge_index, W, b):
    loops = jnp.arange(N, dtype=jnp.int32)
    pad = jnp.full((E_PAD - E_TOT,), DUMP, dtype=jnp.int32)
    row = jnp.concatenate([edge_index[0], loops, pad]).reshape(NW, CPT, CHUNK)
    col = jnp.concatenate([edge_index[1], loops, pad]).reshape(NW, CPT, CHUNK)
    x_pad = jnp.pad(x, ((0, N_PAD - N), (0, 0)))
    z128 = jnp.zeros((N_PAD, D), jnp.float32)
    ones128 = jnp.ones((CHUNK, D), jnp.float32)

    dg = _hist(col, z128, ones128)
    g0 = _scale0(dg, x_pad)
    p1 = _hop(g0, row, col, z128)
    g1 = _mid(dg, p1)
    p2 = _hop(g1, row, col, z128)
    return _final(dg, p2, W, b.reshape(1, D))
